# baseline (speedup 1.0000x reference)
import functools
import jax
import jax.numpy as jnp
from jax import lax
from jax.experimental import pallas as pl
from jax.experimental.pallas import tpu as pltpu
from jax.experimental.pallas import tpu_sc as plsc

NBX = 256
N_NETS = 50000
N_PINS = 200000
NC, NS, L = 2, 16, 16
NW = NC * NS
NNW = 1568
NG = NNW // L
NPV = NNW + 16
PMAX = 16384
CH = 128
PSL = 25024
MAPW = NBX * NBX
MSLICE = MAPW // NS
BIG = 3.0e38


def _sc_body(pin1, netpin, flat, wts, out, npv, wv, pixx, pixy, crdx, crdy,
             sidx, svh, svv, zbuf, pspm, maph, mapv, sem):
    cid = lax.axis_index("c")
    sid = lax.axis_index("s")
    wid = cid * jnp.int32(NS) + sid
    base = pl.multiple_of(wid * jnp.int32(NNW), 8)

    i1 = jnp.int32
    def _zb(i, _):
        zbuf[pl.ds(i * i1(L), L)] = jnp.zeros((L,), jnp.float32)
        return 0
    lax.fori_loop(i1(0), i1(MSLICE // L), _zb, 0)
    msl = sid * i1(MSLICE)
    pltpu.sync_copy(zbuf, maph.at[pl.ds(msl, MSLICE)])
    pltpu.sync_copy(zbuf, mapv.at[pl.ds(msl, MSLICE)])

    psl = pl.multiple_of(sid * i1(PSL), 8)
    for h in range(2):
        po = pl.multiple_of(psl + i1(h * (PSL // 2)), 8)
        pltpu.sync_copy(pin1.at[pl.ds(po, PSL // 2)],
                        crdx.at[pl.ds(0, PSL // 2)])
        pltpu.sync_copy(crdx.at[pl.ds(0, PSL // 2)],
                        pspm.at[pl.ds(po, PSL // 2)])

    pltpu.sync_copy(netpin.at[pl.ds(base, NPV)], npv)
    pltpu.sync_copy(wts.at[pl.ds(base, NNW)], wv)
    s_start = npv[pl.ds(0, L)][0]
    s_end = npv[pl.ds(NNW, L)][0]
    abase = pl.multiple_of((s_start // jnp.int32(8)) * jnp.int32(8), 8)
    off = s_start - abase
    cnt = s_end - s_start

    pltpu.sync_copy(flat.at[pl.ds(abase, PMAX)], pixx)
    nch = lax.div(off + cnt + jnp.int32(CH - 1), jnp.int32(CH))
    nch = jnp.minimum(nch, i1(PMAX // CH))

    def _split(i, _):
        o = i * i1(L)
        f2 = pixx[pl.ds(o, L)] * i1(2)
        pixy[pl.ds(o, L)] = f2 + i1(1)
        pixx[pl.ds(o, L)] = f2
        return 0
    lax.fori_loop(i1(0), nch * i1(CH // L), _split, 0)

    plsc.subcore_barrier()

    FW = 8
    def _fire(i, _):
        o = i * i1(CH)
        pltpu.async_copy(pspm.at[pixx.at[pl.ds(o, CH)]],
                         crdx.at[pl.ds(o, CH)], sem)
        pltpu.async_copy(pspm.at[pixy.at[pl.ds(o, CH)]],
                         crdy.at[pl.ds(o, CH)], sem)
        @pl.when(i >= i1(FW))
        def _():
            od = (i - i1(FW)) * i1(CH)
            pltpu.make_async_copy(pspm.at[pixx.at[pl.ds(od, CH)]],
                                  crdx.at[pl.ds(od, CH)], sem).wait()
            pltpu.make_async_copy(pspm.at[pixy.at[pl.ds(od, CH)]],
                                  crdy.at[pl.ds(od, CH)], sem).wait()
        return 0
    lax.fori_loop(i1(0), nch, _fire, 0)

    def _drain(i, _):
        o = i * i1(CH)
        pltpu.make_async_copy(pspm.at[pixx.at[pl.ds(o, CH)]],
                              crdx.at[pl.ds(o, CH)], sem).wait()
        pltpu.make_async_copy(pspm.at[pixy.at[pl.ds(o, CH)]],
                              crdy.at[pl.ds(o, CH)], sem).wait()
        return 0
    lax.fori_loop(jnp.maximum(nch - i1(FW), i1(0)), nch, _drain, 0)

    iota = lax.iota(jnp.int32, L)

    def _group(g, _):
        gb = g * i1(L)
        sv = npv[pl.ds(gb, L)]
        snv = npv[pl.ds(gb + i1(1), L)]
        npin = snv - sv
        rel = (sv - s_start) + off
        f1 = jnp.float32
        npf = npin.astype(jnp.float32)
        for k in (1, 2, 4, 8):
            pidx = jnp.bitwise_xor(iota, i1(k))
            npf = jnp.maximum(npf, npf.at[pidx].get(mode="promise_in_bounds"))
        mx = npf[0].astype(jnp.int32)

        def _pins(t, c):
            xmn, xmx, ymn, ymx = c
            idx = rel + t
            m = t < npin
            idxc = jnp.clip(idx, i1(0), i1(PMAX - 1))
            px = plsc.load_gather(crdx, [idxc])
            py = plsc.load_gather(crdy, [idxc])
            xmn = jnp.minimum(xmn, jnp.where(m, px, f1(BIG)))
            xmx = jnp.maximum(xmx, jnp.where(m, px, f1(-BIG)))
            ymn = jnp.minimum(ymn, jnp.where(m, py, f1(BIG)))
            ymx = jnp.maximum(ymx, jnp.where(m, py, f1(-BIG)))
            return (xmn, xmx, ymn, ymx)

        big16 = jnp.full((L,), BIG, jnp.float32)
        xmn, xmx, ymn, ymx = lax.fori_loop(
            i1(0), mx, _pins, (big16, -big16, big16, -big16))

        one = jnp.float32(1.0)
        zf = f1(0.0); of = f1(1.0)
        xmn = jnp.clip(xmn, zf, of); xmx = jnp.clip(xmx, zf, of)
        ymn = jnp.clip(ymn, zf, of); ymx = jnp.clip(ymx, zf, of)

        n = npin
        w = jnp.full((L,), 1.0, jnp.float32)
        w = jnp.where(n == 4, f1(1.0828), w)
        w = jnp.where(n == 5, f1(1.1536), w)
        w = jnp.where(n == 6, f1(1.2206), w)
        w = jnp.where(n == 7, f1(1.2823), w)
        w = jnp.where(n == 8, f1(1.3385), w)
        w = jnp.where(n == 9, f1(1.3991), w)
        w = jnp.where(n == 10, f1(1.4493), w)
        w = jnp.where((n >= 11) & (n <= 15), f1(1.6899), w)
        w = jnp.where((n >= 16) & (n <= 20), f1(1.8924), w)
        w = jnp.where((n >= 21) & (n <= 25), f1(2.0743), w)
        w = jnp.where((n >= 26) & (n <= 30), f1(2.2334), w)
        w = jnp.where(n >= 31, f1(2.3892), w)
        wt = w * wv[pl.ds(gb, L)]
        wt = jnp.where(n > 0, wt, zf)

        hx = xmx - xmn
        hy = ymx - ymn
        sc2 = jnp.float32(1.0 / (NBX * NBX))
        wh = jnp.where(hy > 0, (wt * sc2) / jnp.where(hy > 0, hy, one), zf)
        wv_ = jnp.where(hx > 0, (wt * sc2) / jnp.where(hx > 0, hx, one), zf)

        nbf = f1(NBX)
        u = xmn * nbf; v = xmx * nbf
        jx = u.astype(jnp.int32); kx = v.astype(jnp.int32)
        fu = u - jx.astype(jnp.float32); fv = v - kx.astype(jnp.float32)
        uy = ymn * nbf; vy = ymx * nbf
        jy = uy.astype(jnp.int32); ky = vy.astype(jnp.int32)
        gu = uy - jy.astype(jnp.float32); gv = vy - ky.astype(jnp.float32)

        xpos = (jx, jx + i1(1), kx, kx + i1(1))
        xval = (one - fu, fu, fv - one, -fv)
        ypos = (jy, jy + i1(1), ky, ky + i1(1))
        yval = (one - gu, gu, gv - one, -gv)

        ph = jnp.bitwise_and(g, i1(7))
        rbase = ph * i1(2)
        @pl.when(g >= i1(8))
        def _():
            for r in range(2):
                ri = rbase + i1(r)
                pltpu.make_async_copy(svh.at[ri], maph.at[sidx.at[ri]],
                                      sem).wait()
                pltpu.make_async_copy(svv.at[ri], mapv.at[sidx.at[ri]],
                                      sem).wait()
        for dx in range(4):
            for dy in range(4):
                c = dx * 4 + dy
                X = xpos[dx]; Y = ypos[dy]
                ok = (X < NBX) & (Y < NBX)
                a = xval[dx] * yval[dy]
                a = jnp.where(ok, a, zf)
                fl = jnp.clip(X * i1(NBX) + Y, i1(0), i1(MAPW - 1))
                r = c // 8
                col = (c % 8) * L
                ri = rbase + i1(r)
                sidx[ri, pl.ds(col, L)] = fl
                svh[ri, pl.ds(col, L)] = wh * a
                svv[ri, pl.ds(col, L)] = wv_ * a

        for r in range(2):
            ri = rbase + i1(r)
            pltpu.async_copy(svh.at[ri], maph.at[sidx.at[ri]], sem, add=True)
            pltpu.async_copy(svv.at[ri], mapv.at[sidx.at[ri]], sem, add=True)
        return 0

    lax.fori_loop(i1(0), i1(NG), _group, 0)

    def _sdrain(i, _):
        ri = jnp.bitwise_and(i, i1(15))
        pltpu.make_async_copy(svh.at[ri], maph.at[sidx.at[ri]], sem).wait()
        pltpu.make_async_copy(svv.at[ri], mapv.at[sidx.at[ri]], sem).wait()
        return 0
    lax.fori_loop(i1(0), i1(16), _sdrain, 0)

    plsc.subcore_barrier()

    pltpu.sync_copy(maph.at[pl.ds(msl, MSLICE)],
                    out.at[cid, i1(0), pl.ds(msl, MSLICE)])
    pltpu.sync_copy(mapv.at[pl.ds(msl, MSLICE)],
                    out.at[cid, i1(1), pl.ds(msl, MSLICE)])


@jax.jit
def _sc_stage(pin1d, netpin, flat, wts):
    mesh = plsc.VectorSubcoreMesh(core_axis_name="c", subcore_axis_name="s",
                                  num_cores=NC, num_subcores=NS)
    f = pl.kernel(
        _sc_body,
        out_type=jax.ShapeDtypeStruct((NC, 2, MAPW), jnp.float32),
        mesh=mesh,
        compiler_params=pltpu.CompilerParams(needs_layout_passes=False),
        scratch_types=[
            pltpu.VMEM((NPV,), jnp.int32),
            pltpu.VMEM((NNW,), jnp.float32),
            pltpu.VMEM((PMAX,), jnp.int32),
            pltpu.VMEM((PMAX,), jnp.int32),
            pltpu.VMEM((PMAX,), jnp.float32),
            pltpu.VMEM((PMAX,), jnp.float32),
            pltpu.VMEM((16, 128), jnp.int32),
            pltpu.VMEM((16, 128), jnp.float32),
            pltpu.VMEM((16, 128), jnp.float32),
            pltpu.VMEM((MSLICE,), jnp.float32),
            pltpu.VMEM_SHARED((NS * PSL,), jnp.float32),
            pltpu.VMEM_SHARED((MAPW,), jnp.float32),
            pltpu.VMEM_SHARED((MAPW,), jnp.float32),
            pltpu.SemaphoreType.DMA,
        ],
    )
    return f(pin1d, netpin, flat, wts)


def _tc_body(d_ref, dens_ref, h_ref, v_ref):
    d = d_ref[...]
    dh = d[0, 0] + d[1, 0]
    dv = d[0, 1] + d[1, 1]
    r = lax.broadcasted_iota(jnp.int32, (NBX, NBX), 0)
    c = lax.broadcasted_iota(jnp.int32, (NBX, NBX), 1)
    lo = (r >= c).astype(jnp.float32)
    up = (r <= c).astype(jnp.float32)
    h = jnp.dot(lo, jnp.dot(dh, up, preferred_element_type=jnp.float32),
                preferred_element_type=jnp.float32)
    v = jnp.dot(lo, jnp.dot(dv, up, preferred_element_type=jnp.float32),
                preferred_element_type=jnp.float32)
    h_ref[...] = h
    v_ref[...] = v
    dens_ref[...] = jnp.abs(h) + jnp.abs(v)


@jax.jit
def _tc_stage(d):
    out = jax.ShapeDtypeStruct((NBX, NBX), jnp.float32)
    return pl.pallas_call(_tc_body, out_shape=(out, out, out))(d)


def kernel(pin_pos, net_weights, netpin_start, flat_netpin):
    pin1d = jnp.pad(pin_pos.astype(jnp.float32), (0, NS * PSL - 2 * N_PINS))
    np32 = netpin_start.astype(jnp.int32)
    np32 = jnp.pad(np32, (0, (NW - 1) * NNW + NPV - (N_NETS + 1)),
                   mode="edge")
    fl32 = jnp.pad(flat_netpin.astype(jnp.int32), (0, PMAX + 8))
    w32 = jnp.pad(net_weights.astype(jnp.float32), (0, NW * NNW - N_NETS))
    d = _sc_stage(pin1d, np32, fl32, w32)
    dens, h, v = _tc_stage(d.reshape(NC, 2, NBX, NBX))
    return (dens, h, v)


# speedup vs baseline: 1.0016x; 1.0016x over previous
import functools
import jax
import jax.numpy as jnp
from jax import lax
from jax.experimental import pallas as pl
from jax.experimental.pallas import tpu as pltpu
from jax.experimental.pallas import tpu_sc as plsc

NBX = 256
N_NETS = 50000
N_PINS = 200000
NC, NS, L = 2, 16, 16
NW = NC * NS
NNW = 1568
NG = NNW // L
NPV = NNW + 16
PMAX = 16384
CH = 128
PSL = 25024
MAPW = NBX * NBX
MSLICE = MAPW // NS
BIG = 3.0e38


def _sc_body(pin1, netpin, flat, wts, out, npv, wv, pixx, pixy, crdx, crdy,
             sidx, svh, svv, zbuf, pspm, maph, mapv, sem):
    cid = lax.axis_index("c")
    sid = lax.axis_index("s")
    wid = cid * jnp.int32(NS) + sid
    base = pl.multiple_of(wid * jnp.int32(NNW), 8)

    i1 = jnp.int32
    def _zb(i, _):
        zbuf[pl.ds(i * i1(L), L)] = jnp.zeros((L,), jnp.float32)
        return 0
    lax.fori_loop(i1(0), i1(MSLICE // L), _zb, 0)
    msl = sid * i1(MSLICE)
    pltpu.sync_copy(zbuf, maph.at[pl.ds(msl, MSLICE)])
    pltpu.sync_copy(zbuf, mapv.at[pl.ds(msl, MSLICE)])

    psl = pl.multiple_of(sid * i1(PSL), 8)
    for h in range(2):
        po = pl.multiple_of(psl + i1(h * (PSL // 2)), 8)
        pltpu.sync_copy(pin1.at[pl.ds(po, PSL // 2)],
                        crdx.at[pl.ds(0, PSL // 2)])
        pltpu.sync_copy(crdx.at[pl.ds(0, PSL // 2)],
                        pspm.at[pl.ds(po, PSL // 2)])

    pltpu.sync_copy(netpin.at[pl.ds(base, NPV)], npv)
    pltpu.sync_copy(wts.at[pl.ds(base, NNW)], wv)
    s_start = npv[pl.ds(0, L)][0]
    s_end = npv[pl.ds(NNW, L)][0]
    abase = pl.multiple_of((s_start // jnp.int32(8)) * jnp.int32(8), 8)
    off = s_start - abase
    cnt = s_end - s_start

    pltpu.sync_copy(flat.at[pl.ds(abase, PMAX)], pixx)
    nch = lax.div(off + cnt + jnp.int32(CH - 1), jnp.int32(CH))
    nch = jnp.minimum(nch, i1(PMAX // CH))

    def _split(i, _):
        o = i * i1(L)
        f2 = pixx[pl.ds(o, L)] * i1(2)
        pixy[pl.ds(o, L)] = f2 + i1(1)
        pixx[pl.ds(o, L)] = f2
        return 0
    lax.fori_loop(i1(0), nch * i1(CH // L), _split, 0)

    plsc.subcore_barrier()

    FW = 8
    def _fire(i, _):
        o = i * i1(CH)
        pltpu.async_copy(pspm.at[pixx.at[pl.ds(o, CH)]],
                         crdx.at[pl.ds(o, CH)], sem)
        pltpu.async_copy(pspm.at[pixy.at[pl.ds(o, CH)]],
                         crdy.at[pl.ds(o, CH)], sem)
        @pl.when(i >= i1(FW))
        def _():
            od = (i - i1(FW)) * i1(CH)
            pltpu.make_async_copy(pspm.at[pixx.at[pl.ds(od, CH)]],
                                  crdx.at[pl.ds(od, CH)], sem).wait()
            pltpu.make_async_copy(pspm.at[pixy.at[pl.ds(od, CH)]],
                                  crdy.at[pl.ds(od, CH)], sem).wait()
        return 0
    lax.fori_loop(i1(0), nch, _fire, 0)

    def _drain(i, _):
        o = i * i1(CH)
        pltpu.make_async_copy(pspm.at[pixx.at[pl.ds(o, CH)]],
                              crdx.at[pl.ds(o, CH)], sem).wait()
        pltpu.make_async_copy(pspm.at[pixy.at[pl.ds(o, CH)]],
                              crdy.at[pl.ds(o, CH)], sem).wait()
        return 0
    lax.fori_loop(jnp.maximum(nch - i1(FW), i1(0)), nch, _drain, 0)

    iota = lax.iota(jnp.int32, L)

    def _group(g, _):
        gb = g * i1(L)
        sv = npv[pl.ds(gb, L)]
        snv = npv[pl.ds(gb + i1(1), L)]
        npin = snv - sv
        rel = (sv - s_start) + off
        f1 = jnp.float32
        npf = npin.astype(jnp.float32)
        for k in (1, 2, 4, 8):
            pidx = jnp.bitwise_xor(iota, i1(k))
            npf = jnp.maximum(npf, npf.at[pidx].get(mode="promise_in_bounds"))
        mx = npf[0].astype(jnp.int32)

        def _pins(t, c):
            xmn, xmx, ymn, ymx = c
            idx = rel + t
            m = t < npin
            idxc = jnp.clip(idx, i1(0), i1(PMAX - 1))
            px = plsc.load_gather(crdx, [idxc])
            py = plsc.load_gather(crdy, [idxc])
            xmn = jnp.minimum(xmn, jnp.where(m, px, f1(BIG)))
            xmx = jnp.maximum(xmx, jnp.where(m, px, f1(-BIG)))
            ymn = jnp.minimum(ymn, jnp.where(m, py, f1(BIG)))
            ymx = jnp.maximum(ymx, jnp.where(m, py, f1(-BIG)))
            return (xmn, xmx, ymn, ymx)

        big16 = jnp.full((L,), BIG, jnp.float32)
        xmn, xmx, ymn, ymx = lax.fori_loop(
            i1(0), mx, _pins, (big16, -big16, big16, -big16))

        one = jnp.float32(1.0)
        zf = f1(0.0); of = f1(1.0)
        xmn = jnp.clip(xmn, zf, of); xmx = jnp.clip(xmx, zf, of)
        ymn = jnp.clip(ymn, zf, of); ymx = jnp.clip(ymx, zf, of)

        n = npin
        w = jnp.full((L,), 1.0, jnp.float32)
        w = jnp.where(n == 4, f1(1.0828), w)
        w = jnp.where(n == 5, f1(1.1536), w)
        w = jnp.where(n == 6, f1(1.2206), w)
        w = jnp.where(n == 7, f1(1.2823), w)
        w = jnp.where(n == 8, f1(1.3385), w)
        w = jnp.where(n == 9, f1(1.3991), w)
        w = jnp.where(n == 10, f1(1.4493), w)
        w = jnp.where((n >= 11) & (n <= 15), f1(1.6899), w)
        w = jnp.where((n >= 16) & (n <= 20), f1(1.8924), w)
        w = jnp.where((n >= 21) & (n <= 25), f1(2.0743), w)
        w = jnp.where((n >= 26) & (n <= 30), f1(2.2334), w)
        w = jnp.where(n >= 31, f1(2.3892), w)
        wt = w * wv[pl.ds(gb, L)]
        wt = jnp.where(n > 0, wt, zf)

        hx = xmx - xmn
        hy = ymx - ymn
        sc2 = jnp.float32(1.0 / (NBX * NBX))
        wh = jnp.where(hy > 0, (wt * sc2) / jnp.where(hy > 0, hy, one), zf)
        wv_ = jnp.where(hx > 0, (wt * sc2) / jnp.where(hx > 0, hx, one), zf)

        nbf = f1(NBX)
        u = xmn * nbf; v = xmx * nbf
        jx = u.astype(jnp.int32); kx = v.astype(jnp.int32)
        fu = u - jx.astype(jnp.float32); fv = v - kx.astype(jnp.float32)
        uy = ymn * nbf; vy = ymx * nbf
        jy = uy.astype(jnp.int32); ky = vy.astype(jnp.int32)
        gu = uy - jy.astype(jnp.float32); gv = vy - ky.astype(jnp.float32)

        xpos = (jx, jx + i1(1), kx, kx + i1(1))
        xval = (one - fu, fu, fv - one, -fv)
        ypos = (jy, jy + i1(1), ky, ky + i1(1))
        yval = (one - gu, gu, gv - one, -gv)

        ph = jnp.bitwise_and(g, i1(7))
        rbase = ph * i1(2)
        @pl.when(g >= i1(8))
        def _():
            pltpu.make_async_copy(pin1.at[pl.ds(0, 512)],
                                  zbuf.at[pl.ds(0, 512)], sem).wait()
        for dx in range(4):
            for dy in range(4):
                c = dx * 4 + dy
                X = xpos[dx]; Y = ypos[dy]
                ok = (X < NBX) & (Y < NBX)
                a = xval[dx] * yval[dy]
                a = jnp.where(ok, a, zf)
                fl = jnp.clip(X * i1(NBX) + Y, i1(0), i1(MAPW - 1))
                r = c // 8
                col = (c % 8) * L
                ri = rbase + i1(r)
                sidx[ri, pl.ds(col, L)] = fl
                svh[ri, pl.ds(col, L)] = wh * a
                svv[ri, pl.ds(col, L)] = wv_ * a

        for r in range(2):
            ri = rbase + i1(r)
            pltpu.async_copy(svh.at[ri], maph.at[sidx.at[ri]], sem, add=True)
            pltpu.async_copy(svv.at[ri], mapv.at[sidx.at[ri]], sem, add=True)
        return 0

    lax.fori_loop(i1(0), i1(NG), _group, 0)

    def _sdrain(i, _):
        pltpu.make_async_copy(pin1.at[pl.ds(0, 512)],
                              zbuf.at[pl.ds(0, 512)], sem).wait()
        return 0
    lax.fori_loop(i1(0), i1(8), _sdrain, 0)

    plsc.subcore_barrier()

    pltpu.sync_copy(maph.at[pl.ds(msl, MSLICE)],
                    out.at[cid, i1(0), pl.ds(msl, MSLICE)])
    pltpu.sync_copy(mapv.at[pl.ds(msl, MSLICE)],
                    out.at[cid, i1(1), pl.ds(msl, MSLICE)])


@jax.jit
def _sc_stage(pin1d, netpin, flat, wts):
    mesh = plsc.VectorSubcoreMesh(core_axis_name="c", subcore_axis_name="s",
                                  num_cores=NC, num_subcores=NS)
    f = pl.kernel(
        _sc_body,
        out_type=jax.ShapeDtypeStruct((NC, 2, MAPW), jnp.float32),
        mesh=mesh,
        compiler_params=pltpu.CompilerParams(needs_layout_passes=False),
        scratch_types=[
            pltpu.VMEM((NPV,), jnp.int32),
            pltpu.VMEM((NNW,), jnp.float32),
            pltpu.VMEM((PMAX,), jnp.int32),
            pltpu.VMEM((PMAX,), jnp.int32),
            pltpu.VMEM((PMAX,), jnp.float32),
            pltpu.VMEM((PMAX,), jnp.float32),
            pltpu.VMEM((16, 128), jnp.int32),
            pltpu.VMEM((16, 128), jnp.float32),
            pltpu.VMEM((16, 128), jnp.float32),
            pltpu.VMEM((MSLICE,), jnp.float32),
            pltpu.VMEM_SHARED((NS * PSL,), jnp.float32),
            pltpu.VMEM_SHARED((MAPW,), jnp.float32),
            pltpu.VMEM_SHARED((MAPW,), jnp.float32),
            pltpu.SemaphoreType.DMA,
        ],
    )
    return f(pin1d, netpin, flat, wts)


def _tc_body(d_ref, dens_ref, h_ref, v_ref):
    d = d_ref[...]
    dh = d[0, 0] + d[1, 0]
    dv = d[0, 1] + d[1, 1]
    r = lax.broadcasted_iota(jnp.int32, (NBX, NBX), 0)
    c = lax.broadcasted_iota(jnp.int32, (NBX, NBX), 1)
    lo = (r >= c).astype(jnp.float32)
    up = (r <= c).astype(jnp.float32)
    h = jnp.dot(lo, jnp.dot(dh, up, preferred_element_type=jnp.float32),
                preferred_element_type=jnp.float32)
    v = jnp.dot(lo, jnp.dot(dv, up, preferred_element_type=jnp.float32),
                preferred_element_type=jnp.float32)
    h_ref[...] = h
    v_ref[...] = v
    dens_ref[...] = jnp.abs(h) + jnp.abs(v)


@jax.jit
def _tc_stage(d):
    out = jax.ShapeDtypeStruct((NBX, NBX), jnp.float32)
    return pl.pallas_call(_tc_body, out_shape=(out, out, out))(d)


def kernel(pin_pos, net_weights, netpin_start, flat_netpin):
    pin1d = jnp.pad(pin_pos.astype(jnp.float32), (0, NS * PSL - 2 * N_PINS))
    np32 = netpin_start.astype(jnp.int32)
    np32 = jnp.pad(np32, (0, (NW - 1) * NNW + NPV - (N_NETS + 1)),
                   mode="edge")
    fl32 = jnp.pad(flat_netpin.astype(jnp.int32), (0, PMAX + 8))
    w32 = jnp.pad(net_weights.astype(jnp.float32), (0, NW * NNW - N_NETS))
    d = _sc_stage(pin1d, np32, fl32, w32)
    dens, h, v = _tc_stage(d.reshape(NC, 2, NBX, NBX))
    return (dens, h, v)


# speedup vs baseline: 1.0314x; 1.0297x over previous
import functools
import jax
import jax.numpy as jnp
from jax import lax
from jax.experimental import pallas as pl
from jax.experimental.pallas import tpu as pltpu
from jax.experimental.pallas import tpu_sc as plsc

NBX = 256
N_NETS = 50000
N_PINS = 200000
NC, NS, L = 2, 16, 16
NW = NC * NS
NNW = 1568
NG = NNW // L
NPV = NNW + 16
PMAX = 16384
CH = 128
PSL = 25024
MAPW = NBX * NBX
MSLICE = MAPW // NS
BIG = 3.0e38


def _sc_body(pin1, netpin, flat, wts, out, npv, wv, pixx, pixy, crdx, crdy,
             sidx, svh, svv, zbuf, pspm, maph, mapv, maph1, mapv1, sem):
    cid = lax.axis_index("c")
    sid = lax.axis_index("s")
    wid = cid * jnp.int32(NS) + sid
    base = pl.multiple_of(wid * jnp.int32(NNW), 8)

    i1 = jnp.int32
    def _zb(i, _):
        zbuf[pl.ds(i * i1(L), L)] = jnp.zeros((L,), jnp.float32)
        return 0
    lax.fori_loop(i1(0), i1(MSLICE // L), _zb, 0)
    msl = sid * i1(MSLICE)
    pltpu.sync_copy(zbuf, maph.at[pl.ds(msl, MSLICE)])
    pltpu.sync_copy(zbuf, mapv.at[pl.ds(msl, MSLICE)])
    pltpu.sync_copy(zbuf, maph1.at[pl.ds(msl, MSLICE)])
    pltpu.sync_copy(zbuf, mapv1.at[pl.ds(msl, MSLICE)])

    psl = pl.multiple_of(sid * i1(PSL), 8)
    for h in range(2):
        po = pl.multiple_of(psl + i1(h * (PSL // 2)), 8)
        pltpu.sync_copy(pin1.at[pl.ds(po, PSL // 2)],
                        crdx.at[pl.ds(0, PSL // 2)])
        pltpu.sync_copy(crdx.at[pl.ds(0, PSL // 2)],
                        pspm.at[pl.ds(po, PSL // 2)])

    pltpu.sync_copy(netpin.at[pl.ds(base, NPV)], npv)
    pltpu.sync_copy(wts.at[pl.ds(base, NNW)], wv)
    s_start = npv[pl.ds(0, L)][0]
    s_end = npv[pl.ds(NNW, L)][0]
    abase = pl.multiple_of((s_start // jnp.int32(8)) * jnp.int32(8), 8)
    off = s_start - abase
    cnt = s_end - s_start

    pltpu.sync_copy(flat.at[pl.ds(abase, PMAX)], pixx)
    nch = lax.div(off + cnt + jnp.int32(CH - 1), jnp.int32(CH))
    nch = jnp.minimum(nch, i1(PMAX // CH))

    def _split(i, _):
        o = i * i1(L)
        f2 = pixx[pl.ds(o, L)] * i1(2)
        pixy[pl.ds(o, L)] = f2 + i1(1)
        pixx[pl.ds(o, L)] = f2
        return 0
    lax.fori_loop(i1(0), nch * i1(CH // L), _split, 0)

    plsc.subcore_barrier()

    FW = 8
    def _fire(i, _):
        o = i * i1(CH)
        pltpu.async_copy(pspm.at[pixx.at[pl.ds(o, CH)]],
                         crdx.at[pl.ds(o, CH)], sem)
        pltpu.async_copy(pspm.at[pixy.at[pl.ds(o, CH)]],
                         crdy.at[pl.ds(o, CH)], sem)
        @pl.when(i >= i1(FW))
        def _():
            od = (i - i1(FW)) * i1(CH)
            pltpu.make_async_copy(pspm.at[pixx.at[pl.ds(od, CH)]],
                                  crdx.at[pl.ds(od, CH)], sem).wait()
            pltpu.make_async_copy(pspm.at[pixy.at[pl.ds(od, CH)]],
                                  crdy.at[pl.ds(od, CH)], sem).wait()
        return 0
    lax.fori_loop(i1(0), nch, _fire, 0)

    def _drain(i, _):
        o = i * i1(CH)
        pltpu.make_async_copy(pspm.at[pixx.at[pl.ds(o, CH)]],
                              crdx.at[pl.ds(o, CH)], sem).wait()
        pltpu.make_async_copy(pspm.at[pixy.at[pl.ds(o, CH)]],
                              crdy.at[pl.ds(o, CH)], sem).wait()
        return 0
    lax.fori_loop(jnp.maximum(nch - i1(FW), i1(0)), nch, _drain, 0)

    iota = lax.iota(jnp.int32, L)

    def _group(g, _):
        gb = g * i1(L)
        sv = npv[pl.ds(gb, L)]
        snv = npv[pl.ds(gb + i1(1), L)]
        npin = snv - sv
        rel = (sv - s_start) + off
        f1 = jnp.float32
        npf = npin.astype(jnp.float32)
        for k in (1, 2, 4, 8):
            pidx = jnp.bitwise_xor(iota, i1(k))
            npf = jnp.maximum(npf, npf.at[pidx].get(mode="promise_in_bounds"))
        mx = npf[0].astype(jnp.int32)

        def _pins(t, c):
            xmn, xmx, ymn, ymx = c
            idx = rel + t
            m = t < npin
            idxc = jnp.clip(idx, i1(0), i1(PMAX - 1))
            px = plsc.load_gather(crdx, [idxc])
            py = plsc.load_gather(crdy, [idxc])
            xmn = jnp.minimum(xmn, jnp.where(m, px, f1(BIG)))
            xmx = jnp.maximum(xmx, jnp.where(m, px, f1(-BIG)))
            ymn = jnp.minimum(ymn, jnp.where(m, py, f1(BIG)))
            ymx = jnp.maximum(ymx, jnp.where(m, py, f1(-BIG)))
            return (xmn, xmx, ymn, ymx)

        big16 = jnp.full((L,), BIG, jnp.float32)
        xmn, xmx, ymn, ymx = lax.fori_loop(
            i1(0), mx, _pins, (big16, -big16, big16, -big16))

        one = jnp.float32(1.0)
        zf = f1(0.0); of = f1(1.0)
        xmn = jnp.clip(xmn, zf, of); xmx = jnp.clip(xmx, zf, of)
        ymn = jnp.clip(ymn, zf, of); ymx = jnp.clip(ymx, zf, of)

        n = npin
        w = jnp.full((L,), 1.0, jnp.float32)
        w = jnp.where(n == 4, f1(1.0828), w)
        w = jnp.where(n == 5, f1(1.1536), w)
        w = jnp.where(n == 6, f1(1.2206), w)
        w = jnp.where(n == 7, f1(1.2823), w)
        w = jnp.where(n == 8, f1(1.3385), w)
        w = jnp.where(n == 9, f1(1.3991), w)
        w = jnp.where(n == 10, f1(1.4493), w)
        w = jnp.where((n >= 11) & (n <= 15), f1(1.6899), w)
        w = jnp.where((n >= 16) & (n <= 20), f1(1.8924), w)
        w = jnp.where((n >= 21) & (n <= 25), f1(2.0743), w)
        w = jnp.where((n >= 26) & (n <= 30), f1(2.2334), w)
        w = jnp.where(n >= 31, f1(2.3892), w)
        wt = w * wv[pl.ds(gb, L)]
        wt = jnp.where(n > 0, wt, zf)

        hx = xmx - xmn
        hy = ymx - ymn
        sc2 = jnp.float32(1.0 / (NBX * NBX))
        wh = jnp.where(hy > 0, (wt * sc2) / jnp.where(hy > 0, hy, one), zf)
        wv_ = jnp.where(hx > 0, (wt * sc2) / jnp.where(hx > 0, hx, one), zf)

        nbf = f1(NBX)
        u = xmn * nbf; v = xmx * nbf
        jx = u.astype(jnp.int32); kx = v.astype(jnp.int32)
        fu = u - jx.astype(jnp.float32); fv = v - kx.astype(jnp.float32)
        uy = ymn * nbf; vy = ymx * nbf
        jy = uy.astype(jnp.int32); ky = vy.astype(jnp.int32)
        gu = uy - jy.astype(jnp.float32); gv = vy - ky.astype(jnp.float32)

        xpos = (jx, jx + i1(1), kx, kx + i1(1))
        xval = (one - fu, fu, fv - one, -fv)
        ypos = (jy, jy + i1(1), ky, ky + i1(1))
        yval = (one - gu, gu, gv - one, -gv)

        ph = jnp.bitwise_and(g, i1(7))
        rbase = ph * i1(2)
        @pl.when(g >= i1(8))
        def _():
            pltpu.make_async_copy(pin1.at[pl.ds(0, 512)],
                                  zbuf.at[pl.ds(0, 512)], sem).wait()
        for dx in range(4):
            for dy in range(4):
                c = dx * 4 + dy
                X = xpos[dx]; Y = ypos[dy]
                ok = (X < NBX) & (Y < NBX)
                a = xval[dx] * yval[dy]
                a = jnp.where(ok, a, zf)
                fl = jnp.clip(X * i1(NBX) + Y, i1(0), i1(MAPW - 1))
                r = c // 8
                col = (c % 8) * L
                ri = rbase + i1(r)
                sidx[ri, pl.ds(col, L)] = fl
                svh[ri, pl.ds(col, L)] = wh * a
                svv[ri, pl.ds(col, L)] = wv_ * a

        @pl.when(sid < i1(NS // 2))
        def _():
            for r in range(2):
                ri = rbase + i1(r)
                pltpu.async_copy(svh.at[ri], maph.at[sidx.at[ri]], sem,
                                 add=True)
                pltpu.async_copy(svv.at[ri], mapv.at[sidx.at[ri]], sem,
                                 add=True)
        @pl.when(sid >= i1(NS // 2))
        def _():
            for r in range(2):
                ri = rbase + i1(r)
                pltpu.async_copy(svh.at[ri], maph1.at[sidx.at[ri]], sem,
                                 add=True)
                pltpu.async_copy(svv.at[ri], mapv1.at[sidx.at[ri]], sem,
                                 add=True)
        return 0

    lax.fori_loop(i1(0), i1(NG), _group, 0)

    def _sdrain(i, _):
        pltpu.make_async_copy(pin1.at[pl.ds(0, 512)],
                              zbuf.at[pl.ds(0, 512)], sem).wait()
        return 0
    lax.fori_loop(i1(0), i1(8), _sdrain, 0)

    plsc.subcore_barrier()

    for m0, m1, oc in ((maph, maph1, 0), (mapv, mapv1, 1)):
        pltpu.sync_copy(m0.at[pl.ds(msl, MSLICE)], crdx.at[pl.ds(0, MSLICE)])
        pltpu.sync_copy(m1.at[pl.ds(msl, MSLICE)], crdy.at[pl.ds(0, MSLICE)])
        def _mg(i, _):
            o = i * i1(L)
            zbuf[pl.ds(o, L)] = crdx[pl.ds(o, L)] + crdy[pl.ds(o, L)]
            return 0
        lax.fori_loop(i1(0), i1(MSLICE // L), _mg, 0)
        pltpu.sync_copy(zbuf, out.at[cid, i1(oc), pl.ds(msl, MSLICE)])


@jax.jit
def _sc_stage(pin1d, netpin, flat, wts):
    mesh = plsc.VectorSubcoreMesh(core_axis_name="c", subcore_axis_name="s",
                                  num_cores=NC, num_subcores=NS)
    f = pl.kernel(
        _sc_body,
        out_type=jax.ShapeDtypeStruct((NC, 2, MAPW), jnp.float32),
        mesh=mesh,
        compiler_params=pltpu.CompilerParams(needs_layout_passes=False),
        scratch_types=[
            pltpu.VMEM((NPV,), jnp.int32),
            pltpu.VMEM((NNW,), jnp.float32),
            pltpu.VMEM((PMAX,), jnp.int32),
            pltpu.VMEM((PMAX,), jnp.int32),
            pltpu.VMEM((PMAX,), jnp.float32),
            pltpu.VMEM((PMAX,), jnp.float32),
            pltpu.VMEM((16, 128), jnp.int32),
            pltpu.VMEM((16, 128), jnp.float32),
            pltpu.VMEM((16, 128), jnp.float32),
            pltpu.VMEM((MSLICE,), jnp.float32),
            pltpu.VMEM_SHARED((NS * PSL,), jnp.float32),
            pltpu.VMEM_SHARED((MAPW,), jnp.float32),
            pltpu.VMEM_SHARED((MAPW,), jnp.float32),
            pltpu.VMEM_SHARED((MAPW,), jnp.float32),
            pltpu.VMEM_SHARED((MAPW,), jnp.float32),
            pltpu.SemaphoreType.DMA,
        ],
    )
    return f(pin1d, netpin, flat, wts)


def _tc_body(d_ref, dens_ref, h_ref, v_ref):
    d = d_ref[...]
    dh = d[0, 0] + d[1, 0]
    dv = d[0, 1] + d[1, 1]
    r = lax.broadcasted_iota(jnp.int32, (NBX, NBX), 0)
    c = lax.broadcasted_iota(jnp.int32, (NBX, NBX), 1)
    lo = (r >= c).astype(jnp.float32)
    up = (r <= c).astype(jnp.float32)
    h = jnp.dot(lo, jnp.dot(dh, up, preferred_element_type=jnp.float32),
                preferred_element_type=jnp.float32)
    v = jnp.dot(lo, jnp.dot(dv, up, preferred_element_type=jnp.float32),
                preferred_element_type=jnp.float32)
    h_ref[...] = h
    v_ref[...] = v
    dens_ref[...] = jnp.abs(h) + jnp.abs(v)


@jax.jit
def _tc_stage(d):
    out = jax.ShapeDtypeStruct((NBX, NBX), jnp.float32)
    return pl.pallas_call(_tc_body, out_shape=(out, out, out))(d)


def kernel(pin_pos, net_weights, netpin_start, flat_netpin):
    pin1d = jnp.pad(pin_pos.astype(jnp.float32), (0, NS * PSL - 2 * N_PINS))
    np32 = netpin_start.astype(jnp.int32)
    np32 = jnp.pad(np32, (0, (NW - 1) * NNW + NPV - (N_NETS + 1)),
                   mode="edge")
    fl32 = jnp.pad(flat_netpin.astype(jnp.int32), (0, PMAX + 8))
    w32 = jnp.pad(net_weights.astype(jnp.float32), (0, NW * NNW - N_NETS))
    d = _sc_stage(pin1d, np32, fl32, w32)
    dens, h, v = _tc_stage(d.reshape(NC, 2, NBX, NBX))
    return (dens, h, v)


# speedup vs baseline: 1.0766x; 1.0439x over previous
import functools
import jax
import jax.numpy as jnp
from jax import lax
from jax.experimental import pallas as pl
from jax.experimental.pallas import tpu as pltpu
from jax.experimental.pallas import tpu_sc as plsc

NBX = 256
N_NETS = 50000
N_PINS = 200000
NC, NS, L = 2, 16, 16
NW = NC * NS
NNW = 1568
NG = NNW // L
NPV = NNW + 16
PMAX = 16384
CH = 128
PSL = 25024
MAPW = NBX * NBX
MSLICE = MAPW // NS
BIG = 3.0e38


def _sc_body(pin1, netpin, flat, wts, out, npv, wv, pixx, pixy, crdx, crdy,
             sidx, svh, svv, zbuf, pspm, maph, mapv, maph1, mapv1, sem,
             sem2, sem3):
    cid = lax.axis_index("c")
    sid = lax.axis_index("s")
    wid = cid * jnp.int32(NS) + sid
    base = pl.multiple_of(wid * jnp.int32(NNW), 8)

    i1 = jnp.int32
    def _zb(i, _):
        zbuf[pl.ds(i * i1(L), L)] = jnp.zeros((L,), jnp.float32)
        return 0
    lax.fori_loop(i1(0), i1(MSLICE // L), _zb, 0)
    msl = sid * i1(MSLICE)
    psl = pl.multiple_of(sid * i1(PSL), 8)
    po0 = pl.multiple_of(psl, 8)
    po1 = pl.multiple_of(psl + i1(PSL // 2), 8)

    pltpu.async_copy(zbuf, maph.at[pl.ds(msl, MSLICE)], sem2)
    pltpu.async_copy(zbuf, mapv.at[pl.ds(msl, MSLICE)], sem2)
    pltpu.async_copy(zbuf, maph1.at[pl.ds(msl, MSLICE)], sem2)
    pltpu.async_copy(zbuf, mapv1.at[pl.ds(msl, MSLICE)], sem2)
    pltpu.async_copy(pin1.at[pl.ds(po0, PSL // 2)],
                     crdx.at[pl.ds(0, PSL // 2)], sem3)
    pltpu.async_copy(pin1.at[pl.ds(po1, PSL // 2)],
                     crdy.at[pl.ds(0, PSL // 2)], sem3)
    pltpu.sync_copy(netpin.at[pl.ds(base, NPV)], npv)
    pltpu.sync_copy(wts.at[pl.ds(base, NNW)], wv)
    pltpu.make_async_copy(pin1.at[pl.ds(po0, PSL // 2)],
                          crdx.at[pl.ds(0, PSL // 2)], sem3).wait()
    pltpu.make_async_copy(pin1.at[pl.ds(po1, PSL // 2)],
                          crdy.at[pl.ds(0, PSL // 2)], sem3).wait()
    pltpu.async_copy(crdx.at[pl.ds(0, PSL // 2)],
                     pspm.at[pl.ds(po0, PSL // 2)], sem3)
    pltpu.async_copy(crdy.at[pl.ds(0, PSL // 2)],
                     pspm.at[pl.ds(po1, PSL // 2)], sem3)
    s_start = npv[pl.ds(0, L)][0]
    s_end = npv[pl.ds(NNW, L)][0]
    abase = pl.multiple_of((s_start // jnp.int32(8)) * jnp.int32(8), 8)
    off = s_start - abase
    cnt = s_end - s_start

    pltpu.sync_copy(flat.at[pl.ds(abase, PMAX)], pixx)
    nch = lax.div(off + cnt + jnp.int32(CH - 1), jnp.int32(CH))
    nch = jnp.minimum(nch, i1(PMAX // CH))

    def _split(i, _):
        o = i * i1(L)
        f2 = pixx[pl.ds(o, L)] * i1(2)
        pixy[pl.ds(o, L)] = f2 + i1(1)
        pixx[pl.ds(o, L)] = f2
        return 0
    lax.fori_loop(i1(0), nch * i1(CH // L), _split, 0)

    pltpu.make_async_copy(zbuf, maph.at[pl.ds(msl, MSLICE)], sem2).wait()
    pltpu.make_async_copy(zbuf, mapv.at[pl.ds(msl, MSLICE)], sem2).wait()
    pltpu.make_async_copy(zbuf, maph1.at[pl.ds(msl, MSLICE)], sem2).wait()
    pltpu.make_async_copy(zbuf, mapv1.at[pl.ds(msl, MSLICE)], sem2).wait()
    pltpu.make_async_copy(crdx.at[pl.ds(0, PSL // 2)],
                          pspm.at[pl.ds(po0, PSL // 2)], sem3).wait()
    pltpu.make_async_copy(crdy.at[pl.ds(0, PSL // 2)],
                          pspm.at[pl.ds(po1, PSL // 2)], sem3).wait()

    plsc.subcore_barrier()

    FW = 8
    def _fire(i, _):
        o = i * i1(CH)
        pltpu.async_copy(pspm.at[pixx.at[pl.ds(o, CH)]],
                         crdx.at[pl.ds(o, CH)], sem)
        pltpu.async_copy(pspm.at[pixy.at[pl.ds(o, CH)]],
                         crdy.at[pl.ds(o, CH)], sem)
        @pl.when(i >= i1(FW))
        def _():
            od = (i - i1(FW)) * i1(CH)
            pltpu.make_async_copy(pspm.at[pixx.at[pl.ds(od, CH)]],
                                  crdx.at[pl.ds(od, CH)], sem).wait()
            pltpu.make_async_copy(pspm.at[pixy.at[pl.ds(od, CH)]],
                                  crdy.at[pl.ds(od, CH)], sem).wait()
        return 0
    lax.fori_loop(i1(0), nch, _fire, 0)

    def _drain(i, _):
        o = i * i1(CH)
        pltpu.make_async_copy(pspm.at[pixx.at[pl.ds(o, CH)]],
                              crdx.at[pl.ds(o, CH)], sem).wait()
        pltpu.make_async_copy(pspm.at[pixy.at[pl.ds(o, CH)]],
                              crdy.at[pl.ds(o, CH)], sem).wait()
        return 0
    lax.fori_loop(jnp.maximum(nch - i1(FW), i1(0)), nch, _drain, 0)

    iota = lax.iota(jnp.int32, L)

    def _group(g, _):
        gb = g * i1(L)
        sv = npv[pl.ds(gb, L)]
        snv = npv[pl.ds(gb + i1(1), L)]
        npin = snv - sv
        rel = (sv - s_start) + off
        f1 = jnp.float32
        npf = npin.astype(jnp.float32)
        for k in (1, 2, 4, 8):
            pidx = jnp.bitwise_xor(iota, i1(k))
            npf = jnp.maximum(npf, npf.at[pidx].get(mode="promise_in_bounds"))
        mx = npf[0].astype(jnp.int32)

        def _pins(t, c):
            xmn, xmx, ymn, ymx = c
            idx = rel + t
            m = t < npin
            idxc = jnp.clip(idx, i1(0), i1(PMAX - 1))
            px = plsc.load_gather(crdx, [idxc])
            py = plsc.load_gather(crdy, [idxc])
            xmn = jnp.minimum(xmn, jnp.where(m, px, f1(BIG)))
            xmx = jnp.maximum(xmx, jnp.where(m, px, f1(-BIG)))
            ymn = jnp.minimum(ymn, jnp.where(m, py, f1(BIG)))
            ymx = jnp.maximum(ymx, jnp.where(m, py, f1(-BIG)))
            return (xmn, xmx, ymn, ymx)

        big16 = jnp.full((L,), BIG, jnp.float32)
        xmn, xmx, ymn, ymx = lax.fori_loop(
            i1(0), mx, _pins, (big16, -big16, big16, -big16))

        one = jnp.float32(1.0)
        zf = f1(0.0); of = f1(1.0)
        xmn = jnp.clip(xmn, zf, of); xmx = jnp.clip(xmx, zf, of)
        ymn = jnp.clip(ymn, zf, of); ymx = jnp.clip(ymx, zf, of)

        n = npin
        w = jnp.full((L,), 1.0, jnp.float32)
        w = jnp.where(n == 4, f1(1.0828), w)
        w = jnp.where(n == 5, f1(1.1536), w)
        w = jnp.where(n == 6, f1(1.2206), w)
        w = jnp.where(n == 7, f1(1.2823), w)
        w = jnp.where(n == 8, f1(1.3385), w)
        w = jnp.where(n == 9, f1(1.3991), w)
        w = jnp.where(n == 10, f1(1.4493), w)
        w = jnp.where((n >= 11) & (n <= 15), f1(1.6899), w)
        w = jnp.where((n >= 16) & (n <= 20), f1(1.8924), w)
        w = jnp.where((n >= 21) & (n <= 25), f1(2.0743), w)
        w = jnp.where((n >= 26) & (n <= 30), f1(2.2334), w)
        w = jnp.where(n >= 31, f1(2.3892), w)
        wt = w * wv[pl.ds(gb, L)]
        wt = jnp.where(n > 0, wt, zf)

        hx = xmx - xmn
        hy = ymx - ymn
        sc2 = jnp.float32(1.0 / (NBX * NBX))
        wh = jnp.where(hy > 0, (wt * sc2) / jnp.where(hy > 0, hy, one), zf)
        wv_ = jnp.where(hx > 0, (wt * sc2) / jnp.where(hx > 0, hx, one), zf)

        nbf = f1(NBX)
        u = xmn * nbf; v = xmx * nbf
        jx = u.astype(jnp.int32); kx = v.astype(jnp.int32)
        fu = u - jx.astype(jnp.float32); fv = v - kx.astype(jnp.float32)
        uy = ymn * nbf; vy = ymx * nbf
        jy = uy.astype(jnp.int32); ky = vy.astype(jnp.int32)
        gu = uy - jy.astype(jnp.float32); gv = vy - ky.astype(jnp.float32)

        xpos = (jx, jx + i1(1), kx, kx + i1(1))
        xval = (one - fu, fu, fv - one, -fv)
        ypos = (jy, jy + i1(1), ky, ky + i1(1))
        yval = (one - gu, gu, gv - one, -gv)

        ph = jnp.bitwise_and(g, i1(7))
        rbase = ph * i1(2)
        @pl.when(g >= i1(8))
        def _():
            pltpu.make_async_copy(pin1.at[pl.ds(0, 512)],
                                  zbuf.at[pl.ds(0, 512)], sem).wait()
        for dx in range(4):
            for dy in range(4):
                c = dx * 4 + dy
                X = xpos[dx]; Y = ypos[dy]
                ok = (X < NBX) & (Y < NBX)
                a = xval[dx] * yval[dy]
                a = jnp.where(ok, a, zf)
                fl = jnp.clip(X * i1(NBX) + Y, i1(0), i1(MAPW - 1))
                r = c // 8
                col = (c % 8) * L
                ri = rbase + i1(r)
                sidx[ri, pl.ds(col, L)] = fl
                svh[ri, pl.ds(col, L)] = wh * a
                svv[ri, pl.ds(col, L)] = wv_ * a

        @pl.when(sid < i1(NS // 2))
        def _():
            for r in range(2):
                ri = rbase + i1(r)
                pltpu.async_copy(svh.at[ri], maph.at[sidx.at[ri]], sem,
                                 add=True)
                pltpu.async_copy(svv.at[ri], mapv.at[sidx.at[ri]], sem,
                                 add=True)
        @pl.when(sid >= i1(NS // 2))
        def _():
            for r in range(2):
                ri = rbase + i1(r)
                pltpu.async_copy(svh.at[ri], maph1.at[sidx.at[ri]], sem,
                                 add=True)
                pltpu.async_copy(svv.at[ri], mapv1.at[sidx.at[ri]], sem,
                                 add=True)
        return 0

    lax.fori_loop(i1(0), i1(NG), _group, 0)

    def _sdrain(i, _):
        pltpu.make_async_copy(pin1.at[pl.ds(0, 512)],
                              zbuf.at[pl.ds(0, 512)], sem).wait()
        return 0
    lax.fori_loop(i1(0), i1(8), _sdrain, 0)

    plsc.subcore_barrier()

    for m0, m1, oc in ((maph, maph1, 0), (mapv, mapv1, 1)):
        pltpu.sync_copy(m0.at[pl.ds(msl, MSLICE)], crdx.at[pl.ds(0, MSLICE)])
        pltpu.sync_copy(m1.at[pl.ds(msl, MSLICE)], crdy.at[pl.ds(0, MSLICE)])
        def _mg(i, _):
            o = i * i1(L)
            zbuf[pl.ds(o, L)] = crdx[pl.ds(o, L)] + crdy[pl.ds(o, L)]
            return 0
        lax.fori_loop(i1(0), i1(MSLICE // L), _mg, 0)
        pltpu.sync_copy(zbuf, out.at[cid, i1(oc), pl.ds(msl, MSLICE)])


@jax.jit
def _sc_stage(pin1d, netpin, flat, wts):
    mesh = plsc.VectorSubcoreMesh(core_axis_name="c", subcore_axis_name="s",
                                  num_cores=NC, num_subcores=NS)
    f = pl.kernel(
        _sc_body,
        out_type=jax.ShapeDtypeStruct((NC, 2, MAPW), jnp.float32),
        mesh=mesh,
        compiler_params=pltpu.CompilerParams(needs_layout_passes=False),
        scratch_types=[
            pltpu.VMEM((NPV,), jnp.int32),
            pltpu.VMEM((NNW,), jnp.float32),
            pltpu.VMEM((PMAX,), jnp.int32),
            pltpu.VMEM((PMAX,), jnp.int32),
            pltpu.VMEM((PMAX,), jnp.float32),
            pltpu.VMEM((PMAX,), jnp.float32),
            pltpu.VMEM((16, 128), jnp.int32),
            pltpu.VMEM((16, 128), jnp.float32),
            pltpu.VMEM((16, 128), jnp.float32),
            pltpu.VMEM((MSLICE,), jnp.float32),
            pltpu.VMEM_SHARED((NS * PSL,), jnp.float32),
            pltpu.VMEM_SHARED((MAPW,), jnp.float32),
            pltpu.VMEM_SHARED((MAPW,), jnp.float32),
            pltpu.VMEM_SHARED((MAPW,), jnp.float32),
            pltpu.VMEM_SHARED((MAPW,), jnp.float32),
            pltpu.SemaphoreType.DMA,
            pltpu.SemaphoreType.DMA,
            pltpu.SemaphoreType.DMA,
        ],
    )
    return f(pin1d, netpin, flat, wts)


def _tc_body(d_ref, dens_ref, h_ref, v_ref):
    d = d_ref[...]
    dh = d[0, 0] + d[1, 0]
    dv = d[0, 1] + d[1, 1]
    r = lax.broadcasted_iota(jnp.int32, (NBX, NBX), 0)
    c = lax.broadcasted_iota(jnp.int32, (NBX, NBX), 1)
    lo = (r >= c).astype(jnp.float32)
    up = (r <= c).astype(jnp.float32)
    h = jnp.dot(lo, jnp.dot(dh, up, preferred_element_type=jnp.float32),
                preferred_element_type=jnp.float32)
    v = jnp.dot(lo, jnp.dot(dv, up, preferred_element_type=jnp.float32),
                preferred_element_type=jnp.float32)
    h_ref[...] = h
    v_ref[...] = v
    dens_ref[...] = jnp.abs(h) + jnp.abs(v)


@jax.jit
def _tc_stage(d):
    out = jax.ShapeDtypeStruct((NBX, NBX), jnp.float32)
    return pl.pallas_call(_tc_body, out_shape=(out, out, out))(d)


def kernel(pin_pos, net_weights, netpin_start, flat_netpin):
    pin1d = jnp.pad(pin_pos.astype(jnp.float32), (0, NS * PSL - 2 * N_PINS))
    np32 = netpin_start.astype(jnp.int32)
    np32 = jnp.pad(np32, (0, (NW - 1) * NNW + NPV - (N_NETS + 1)),
                   mode="edge")
    fl32 = jnp.pad(flat_netpin.astype(jnp.int32), (0, PMAX + 8))
    w32 = jnp.pad(net_weights.astype(jnp.float32), (0, NW * NNW - N_NETS))
    d = _sc_stage(pin1d, np32, fl32, w32)
    dens, h, v = _tc_stage(d.reshape(NC, 2, NBX, NBX))
    return (dens, h, v)
